# baseline (speedup 1.0000x reference)
import functools

import jax
import jax.numpy as jnp
from jax import lax
from jax.experimental import pallas as pl
from jax.experimental.pallas import tpu as pltpu
from jax.experimental.pallas import tpu_sc as plsc

DIM = 64
N_EMBED = 1024
POS_DIM = 16
POS_EMBED = 128
TOKENS = 16384
BLOCK = 1024
GRID = TOKENS // BLOCK


def _tc_body(x_ref, emb_ref, gate_ref, ind_ref, eind_ref, diff_ref, cb_ref):
    b = pl.program_id(0)
    x = x_ref[...]
    emb = emb_ref[...]

    row = lax.broadcasted_iota(jnp.int32, (DIM, N_EMBED), 0)
    col = lax.broadcasted_iota(jnp.int32, (DIM, N_EMBED), 1)
    mask = ((row < DIM - POS_DIM) == (col < N_EMBED - POS_EMBED)).astype(
        jnp.float32)
    emb_masked = emb * mask
    cb = emb_masked * gate_ref[0, 0]

    @pl.when(b == 0)
    def _():
        cb_ref[...] = emb_masked

    xcb = jnp.dot(x, cb, preferred_element_type=jnp.float32)
    cnorm = jnp.sum(cb * cb, axis=0, keepdims=True)
    s = cnorm - 2.0 * xcb
    mins = jnp.min(s, axis=1, keepdims=True)
    onehot = (s <= mins).astype(jnp.float32)
    iota = lax.broadcasted_iota(jnp.int32, (1, N_EMBED), 1).astype(jnp.float32)
    ind = lax.dot_general(iota, onehot, (((1,), (1,)), ((), ())),
                          precision=lax.Precision.HIGHEST,
                          preferred_element_type=jnp.float32)
    ind = jnp.minimum(ind, jnp.float32(N_EMBED - 1))
    ind_i = ind.astype(jnp.int32)
    ind_ref[...] = ind_i.reshape(1, 1, N_EMBED)
    eind_ref[...] = ind_i.reshape(1, 32, 32)

    fnorm = jnp.sum(x * x, axis=1, keepdims=True)
    diff_ref[...] = ((jnp.sum(fnorm) + jnp.sum(mins))
                     / jnp.float32(BLOCK * DIM)).reshape(1, 1, 1)


def _tc_stage(flat, embed, gate):
    return pl.pallas_call(
        _tc_body,
        grid=(GRID,),
        in_specs=[
            pl.BlockSpec((BLOCK, DIM), lambda b: (b, 0)),
            pl.BlockSpec((DIM, N_EMBED), lambda b: (0, 0)),
            pl.BlockSpec((1, 1), lambda b: (0, 0)),
        ],
        out_specs=[
            pl.BlockSpec((1, 1, N_EMBED), lambda b: (b, 0, 0)),
            pl.BlockSpec((1, 32, 32), lambda b: (b, 0, 0)),
            pl.BlockSpec((1, 1, 1), lambda b: (b, 0, 0)),
            pl.BlockSpec((DIM, N_EMBED), lambda b: (0, 0)),
        ],
        out_shape=[
            jax.ShapeDtypeStruct((GRID, 1, N_EMBED), jnp.int32),
            jax.ShapeDtypeStruct((GRID, 32, 32), jnp.int32),
            jax.ShapeDtypeStruct((GRID, 1, 1), jnp.float32),
            jax.ShapeDtypeStruct((DIM, N_EMBED), jnp.float32),
        ],
    )(flat, embed, gate)


_CHUNK = 128


def _sc_gather(table, idx2d):
    info = plsc.get_sparse_core_info()
    _NC, _NS = info.num_cores, info.num_subcores
    _NW = _NC * _NS
    _BPW = TOKENS // _NW
    _NCHUNK = _BPW // _CHUNK
    mesh = plsc.VectorSubcoreMesh(core_axis_name="c", subcore_axis_name="s")

    @functools.partial(
        pl.kernel,
        mesh=mesh,
        compiler_params=pltpu.CompilerParams(use_tc_tiling_on_sc=False),
        out_type=jax.ShapeDtypeStruct((TOKENS, DIM), jnp.float32),
        scratch_types=[
            pltpu.VMEM((_NCHUNK, _CHUNK), jnp.int32),
            pltpu.VMEM((_BPW, DIM), jnp.float32),
            pltpu.SemaphoreType.DMA,
        ],
    )
    def k(table_hbm, idx_hbm, out_hbm, idx_v, rows_v, sem):
        wid = lax.axis_index("s") * _NC + lax.axis_index("c")
        pltpu.sync_copy(idx_hbm.at[pl.ds(wid * _NCHUNK, _NCHUNK), :], idx_v)
        copies = [
            pltpu.async_copy(
                table_hbm.at[idx_v.at[j]],
                rows_v.at[pl.ds(j * _CHUNK, _CHUNK), :],
                sem,
            )
            for j in range(_NCHUNK)
        ]
        for c in copies:
            c.wait()
        pltpu.sync_copy(rows_v, out_hbm.at[pl.ds(wid * _BPW, _BPW), :])

    return k(table, idx2d)


def kernel(input, embed, bi):
    flat = input.reshape(TOKENS, DIM)
    gate = (jnp.asarray(bi) == 1).astype(jnp.float32).reshape(1, 1)
    ind, embed_ind, diff, cb = _tc_stage(flat, embed, gate)
    q = _sc_gather(embed.T, ind.reshape(TOKENS // _CHUNK, _CHUNK))
    ind_flat = ind.reshape(TOKENS)
    keep = ((lax.broadcasted_iota(jnp.int32, (TOKENS, DIM), 1)
             < DIM - POS_DIM)
            == (ind_flat < N_EMBED - POS_EMBED)[:, None])
    quantize = (jnp.where(keep, q, 0.0) * gate[0, 0]).reshape(input.shape)
    return quantize, diff.reshape(GRID), embed_ind, cb


# speedup vs baseline: 1.5282x; 1.5282x over previous
import functools

import jax
import jax.numpy as jnp
from jax import lax
from jax.experimental import pallas as pl
from jax.experimental.pallas import tpu as pltpu
from jax.experimental.pallas import tpu_sc as plsc

DIM = 64
N_EMBED = 1024
POS_DIM = 16
POS_EMBED = 128
TOKENS = 16384
BLOCK = 1024
GRID = TOKENS // BLOCK


def _tc_body(x_ref, emb_ref, gate_ref, ind_ref, eind_ref, diff_ref, cb_ref):
    b = pl.program_id(0)
    x = x_ref[...]
    emb = emb_ref[...]

    row = lax.broadcasted_iota(jnp.int32, (DIM, N_EMBED), 0)
    col = lax.broadcasted_iota(jnp.int32, (DIM, N_EMBED), 1)
    mask = ((row < DIM - POS_DIM) == (col < N_EMBED - POS_EMBED)).astype(
        jnp.float32)
    emb_masked = emb * mask
    cb = emb_masked * gate_ref[0, 0]

    @pl.when(b == 0)
    def _():
        cb_ref[...] = emb_masked

    cbsq = cb * cb
    ones = jnp.full((DIM, 1), 1.0, dtype=jnp.float32)
    cnorm_c = lax.dot_general(cbsq, ones, (((0,), (0,)), ((), ())),
                              preferred_element_type=jnp.float32)
    xcb_t = lax.dot_general(cb, x, (((0,), (1,)), ((), ())),
                            preferred_element_type=jnp.float32)
    st = cnorm_c - 2.0 * xcb_t
    mins = jnp.min(st, axis=0, keepdims=True)
    onehot = (st <= mins).astype(jnp.float32)
    r2 = lax.broadcasted_iota(jnp.int32, (2, N_EMBED), 0)
    j2 = lax.broadcasted_iota(jnp.int32, (2, N_EMBED), 1)
    w2 = jnp.where(r2 == 0, j2 // 8, j2 % 8).astype(jnp.float32)
    hl = lax.dot_general(w2, onehot, (((1,), (0,)), ((), ())),
                         preferred_element_type=jnp.float32)
    ind = 8.0 * lax.slice(hl, (0, 0), (1, BLOCK)) + lax.slice(hl, (1, 0),
                                                              (2, BLOCK))
    ind = jnp.minimum(ind, jnp.float32(N_EMBED - 1))
    ind_i = ind.astype(jnp.int32)
    ind_ref[...] = ind_i.reshape(1, 1, N_EMBED)
    eind_ref[...] = ind_i.reshape(1, 32, 32)

    fnorm = jnp.sum(x * x, axis=1, keepdims=True)
    diff_ref[...] = ((jnp.sum(fnorm) + jnp.sum(mins))
                     / jnp.float32(BLOCK * DIM)).reshape(1, 1, 1)


def _tc_stage(flat, embed, gate):
    return pl.pallas_call(
        _tc_body,
        grid=(GRID,),
        in_specs=[
            pl.BlockSpec((BLOCK, DIM), lambda b: (b, 0)),
            pl.BlockSpec((DIM, N_EMBED), lambda b: (0, 0)),
            pl.BlockSpec((1, 1), lambda b: (0, 0)),
        ],
        out_specs=[
            pl.BlockSpec((1, 1, N_EMBED), lambda b: (b, 0, 0)),
            pl.BlockSpec((1, 32, 32), lambda b: (b, 0, 0)),
            pl.BlockSpec((1, 1, 1), lambda b: (b, 0, 0)),
            pl.BlockSpec((DIM, N_EMBED), lambda b: (0, 0)),
        ],
        out_shape=[
            jax.ShapeDtypeStruct((GRID, 1, N_EMBED), jnp.int32),
            jax.ShapeDtypeStruct((GRID, 32, 32), jnp.int32),
            jax.ShapeDtypeStruct((GRID, 1, 1), jnp.float32),
            jax.ShapeDtypeStruct((DIM, N_EMBED), jnp.float32),
        ],
    )(flat, embed, gate)


_CHUNK = 128


def _sc_gather(table, idx2d):
    info = plsc.get_sparse_core_info()
    _NC, _NS = info.num_cores, info.num_subcores
    _NW = _NC * _NS
    _BPW = TOKENS // _NW
    _NCHUNK = _BPW // _CHUNK
    mesh = plsc.VectorSubcoreMesh(core_axis_name="c", subcore_axis_name="s")

    @functools.partial(
        pl.kernel,
        mesh=mesh,
        compiler_params=pltpu.CompilerParams(use_tc_tiling_on_sc=False),
        out_type=jax.ShapeDtypeStruct((TOKENS, DIM), jnp.float32),
        scratch_types=[
            pltpu.VMEM((_NCHUNK, _CHUNK), jnp.int32),
            pltpu.VMEM((_BPW, DIM), jnp.float32),
            pltpu.SemaphoreType.DMA,
        ],
    )
    def k(table_hbm, idx_hbm, out_hbm, idx_v, rows_v, sem):
        wid = lax.axis_index("s") * _NC + lax.axis_index("c")
        pltpu.sync_copy(idx_hbm.at[pl.ds(wid * _NCHUNK, _NCHUNK), :], idx_v)
        copies = [
            pltpu.async_copy(
                table_hbm.at[idx_v.at[j]],
                rows_v.at[pl.ds(j * _CHUNK, _CHUNK), :],
                sem,
            )
            for j in range(_NCHUNK)
        ]
        for c in copies:
            c.wait()
        pltpu.sync_copy(rows_v, out_hbm.at[pl.ds(wid * _BPW, _BPW), :])

    return k(table, idx2d)


def kernel(input, embed, bi):
    flat = input.reshape(TOKENS, DIM)
    gate = (jnp.asarray(bi) == 1).astype(jnp.float32).reshape(1, 1)
    ind, embed_ind, diff, cb = _tc_stage(flat, embed, gate)
    q = _sc_gather(embed.T, ind.reshape(TOKENS // _CHUNK, _CHUNK))
    ind_flat = ind.reshape(TOKENS)
    keep = ((lax.broadcasted_iota(jnp.int32, (TOKENS, DIM), 1)
             < DIM - POS_DIM)
            == (ind_flat < N_EMBED - POS_EMBED)[:, None])
    quantize = (jnp.where(keep, q, 0.0) * gate[0, 0]).reshape(input.shape)
    return quantize, diff.reshape(GRID), embed_ind, cb
